# Initial kernel scaffold; baseline (speedup 1.0000x reference)
import jax
import jax.numpy as jnp
from jax.experimental import pallas as pl


def kernel(pos, normal, batch, W1_0, b1_0, W2_0, b2_0, W1_1, b1_1, W2_1, b2_1, W1_2, b1_2, W2_2, b2_2, Wc, bc):
    raise NotImplementedError("write your pallas kernel here")


# speedup vs baseline: 1.6554x; 1.6554x over previous
import functools

import jax
import jax.numpy as jnp
from jax import lax
from jax.experimental import pallas as pl
from jax.experimental.pallas import tpu as pltpu
from jax.experimental.pallas import tpu_sc as plsc

K = 16
G = 8
RB = 128
CT = 512
SC_NC = 2
SC_NS = 16
NW = SC_NC * SC_NS

_BIGI = 2**30


def _knn_body(nt, pn_ref, pnT_ref, out_ref):
    i = pl.program_id(0)
    rb = pn_ref[:, :]
    px, py, pz = rb[:, 0:1], rb[:, 1:2], rb[:, 2:3]
    bfr, sqr = rb[:, 6:7], rb[:, 7:8]
    row_ids = lax.broadcasted_iota(jnp.int32, (RB, 1), 0) + i * RB
    inf = jnp.float32(jnp.inf)

    def tile_body(t, carry):
        run_val, run_idx = carry
        c0 = pl.multiple_of(t * CT, CT)
        ct = pnT_ref[:, pl.ds(c0, CT)]
        cx, cy, cz = ct[0:1, :], ct[1:2, :], ct[2:3, :]
        bfc, sqc = ct[6:7, :], ct[7:8, :]
        d = sqr + sqc - 2.0 * (px * cx + py * cy + pz * cz)
        col_ids = lax.broadcasted_iota(jnp.int32, (1, CT), 1) + c0
        bad = (bfr != bfc) | (row_ids == col_ids)
        d = jnp.where(bad, inf, d)
        ev = jnp.concatenate([run_val, d], axis=1)
        ei = jnp.concatenate(
            [run_idx, jnp.broadcast_to(col_ids, d.shape)], axis=1)
        vals, idxs = [], []
        for _ in range(K):
            m = jnp.min(ev, axis=1, keepdims=True)
            pick = ev == m
            sel = jnp.min(jnp.where(pick, ei, _BIGI), axis=1,
                          keepdims=True)
            vals.append(m)
            idxs.append(sel)
            ev = jnp.where(pick & (ei == sel), inf, ev)
        return (jnp.concatenate(vals, axis=1),
                jnp.concatenate(idxs, axis=1))

    init = (jnp.full((RB, K), inf, jnp.float32),
            jnp.full((RB, K), _BIGI, jnp.int32))
    _, run_idx = lax.fori_loop(0, nt, tile_body, init)
    out_ref[:, :] = run_idx


def _knn(pn, pnT):
    npad = pn.shape[0]
    nb = npad // RB
    nt = npad // CT
    return pl.pallas_call(
        functools.partial(_knn_body, nt),
        grid=(nb,),
        in_specs=[
            pl.BlockSpec((RB, 16), lambda i: (i, 0)),
            pl.BlockSpec((16, npad), lambda i: (0, 0)),
        ],
        out_specs=pl.BlockSpec((RB, K), lambda i: (i, 0)),
        out_shape=jax.ShapeDtypeStruct((npad, K), jnp.int32),
    )(pn, pnT)


def _sc_gather(table, idx3):
    nw, nc, lanes = idx3.shape
    d = table.shape[1]
    b = nw * nc * lanes
    mesh = plsc.VectorSubcoreMesh(core_axis_name="c", subcore_axis_name="s")

    @functools.partial(
        pl.kernel,
        out_type=jax.ShapeDtypeStruct((b, d), jnp.float32),
        mesh=mesh,
        scratch_types=[
            pltpu.VMEM((nc, lanes), jnp.int32),
            pltpu.VMEM((lanes, d), jnp.float32),
            pltpu.SemaphoreType.DMA,
        ],
    )
    def gather_kernel(table_hbm, idx_hbm, out_hbm, idx_v, rows_v, sem):
        wid = lax.axis_index("s") * SC_NC + lax.axis_index("c")
        pltpu.sync_copy(idx_hbm.at[wid], idx_v)
        base = wid * (nc * lanes)

        def chunk(c, carry):
            pltpu.async_copy(table_hbm.at[idx_v.at[c]], rows_v, sem).wait()
            pltpu.sync_copy(rows_v, out_hbm.at[pl.ds(base + c * lanes, lanes)])
            return carry

        lax.fori_loop(0, nc, chunk, 0)

    return gather_kernel(table, idx3)


def _ppf_features(dstv, srcv):
    dx = srcv[:, 0:1] - dstv[:, 0:1]
    dy = srcv[:, 1:2] - dstv[:, 1:2]
    dz = srcv[:, 2:3] - dstv[:, 2:3]
    ndx, ndy, ndz = dstv[:, 3:4], dstv[:, 4:5], dstv[:, 5:6]
    nsx, nsy, nsz = srcv[:, 3:4], srcv[:, 4:5], srcv[:, 5:6]

    def ang(ax, ay, az, bx, by, bz):
        cx = ay * bz - az * by
        cy = az * bx - ax * bz
        cz = ax * by - ay * bx
        cn = jnp.sqrt(cx * cx + cy * cy + cz * cz)
        dt = ax * bx + ay * by + az * bz
        return jnp.arctan2(cn, dt)

    f1 = jnp.sqrt(dx * dx + dy * dy + dz * dz)
    f2 = ang(ndx, ndy, ndz, dx, dy, dz)
    f3 = ang(nsx, nsy, nsz, dx, dy, dz)
    f4 = ang(ndx, ndy, ndz, nsx, nsy, nsz)
    return jnp.concatenate([f1, f2, f3, f4], axis=1)


def _ppf_mm(ppf, w1p):
    acc = ppf[:, 0:1] * w1p[0:1, :]
    for c in range(1, 4):
        acc = acc + ppf[:, c:c + 1] * w1p[c:c + 1, :]
    return acc


def _layer0_body(pn_ref, png_ref, w1p_ref, b1_ref, w2_ref, b2_ref,
                 x_ref, ppf_ref):
    j = pl.program_id(1)
    ppf = _ppf_features(pn_ref[:, :], png_ref[0])
    ppf_ref[0] = ppf
    h1 = jnp.maximum(_ppf_mm(ppf, w1p_ref[:, :]) + b1_ref[:, :], 0.0)
    h = jnp.dot(h1, w2_ref[:, :], preferred_element_type=jnp.float32) + b2_ref[:, :]

    @pl.when(j == 0)
    def _():
        hs = (jnp.dot(jnp.maximum(b1_ref[:, :], 0.0), w2_ref[:, :],
                      preferred_element_type=jnp.float32) + b2_ref[:, :])
        x_ref[:, :] = jnp.maximum(h, hs)

    @pl.when(j > 0)
    def _():
        x_ref[:, :] = jnp.maximum(x_ref[:, :], h)

    @pl.when(j == K - 1)
    def _():
        x_ref[:, :] = jnp.maximum(x_ref[:, :], 0.0)


def _layer0(pn, png, w1p, b1, w2, b2):
    npad = pn.shape[0]
    nb = npad // RB
    h = w2.shape[0]
    return pl.pallas_call(
        _layer0_body,
        grid=(nb, K),
        in_specs=[
            pl.BlockSpec((RB, 16), lambda i, j: (i, 0)),
            pl.BlockSpec((1, RB, 128), lambda i, j: (j, i, 0)),
            pl.BlockSpec((4, h), lambda i, j: (0, 0)),
            pl.BlockSpec((1, h), lambda i, j: (0, 0)),
            pl.BlockSpec((h, h), lambda i, j: (0, 0)),
            pl.BlockSpec((1, h), lambda i, j: (0, 0)),
        ],
        out_specs=[
            pl.BlockSpec((RB, h), lambda i, j: (i, 0)),
            pl.BlockSpec((1, RB, 4), lambda i, j: (j, i, 0)),
        ],
        out_shape=[
            jax.ShapeDtypeStruct((npad, h), jnp.float32),
            jax.ShapeDtypeStruct((K, npad, 4), jnp.float32),
        ],
    )(pn, png, w1p, b1, w2, b2)


def _layer_body(x_ref, xg_ref, ppf_ref, w1x_ref, w1p_ref, b1_ref, w2_ref,
                b2_ref, o_ref):
    j = pl.program_id(1)
    h1 = jnp.maximum(
        jnp.dot(xg_ref[0], w1x_ref[:, :], preferred_element_type=jnp.float32)
        + _ppf_mm(ppf_ref[0], w1p_ref[:, :]) + b1_ref[:, :], 0.0)
    h = jnp.dot(h1, w2_ref[:, :], preferred_element_type=jnp.float32) + b2_ref[:, :]

    @pl.when(j == 0)
    def _():
        h1s = jnp.maximum(
            jnp.dot(x_ref[:, :], w1x_ref[:, :],
                    preferred_element_type=jnp.float32) + b1_ref[:, :], 0.0)
        hs = jnp.dot(h1s, w2_ref[:, :], preferred_element_type=jnp.float32) + b2_ref[:, :]
        o_ref[:, :] = jnp.maximum(h, hs)

    @pl.when(j > 0)
    def _():
        o_ref[:, :] = jnp.maximum(o_ref[:, :], h)

    @pl.when(j == K - 1)
    def _():
        o_ref[:, :] = jnp.maximum(o_ref[:, :], 0.0)


def _layer(x, xg, ppfT, w1x, w1p, b1, w2, b2):
    npad = x.shape[0]
    nb = npad // RB
    h = w2.shape[0]
    return pl.pallas_call(
        _layer_body,
        grid=(nb, K),
        in_specs=[
            pl.BlockSpec((RB, h), lambda i, j: (i, 0)),
            pl.BlockSpec((1, RB, h), lambda i, j: (j, i, 0)),
            pl.BlockSpec((1, RB, 4), lambda i, j: (j, i, 0)),
            pl.BlockSpec((h, h), lambda i, j: (0, 0)),
            pl.BlockSpec((4, h), lambda i, j: (0, 0)),
            pl.BlockSpec((1, h), lambda i, j: (0, 0)),
            pl.BlockSpec((h, h), lambda i, j: (0, 0)),
            pl.BlockSpec((1, h), lambda i, j: (0, 0)),
        ],
        out_specs=pl.BlockSpec((RB, h), lambda i, j: (i, 0)),
        out_shape=jax.ShapeDtypeStruct((npad, h), jnp.float32),
    )(x, xg, ppfT, w1x, w1p, b1, w2, b2)


def _pool_body(nb, x_ref, pn_ref, wc_ref, bc_ref, o_ref, pool_ref):
    i = pl.program_id(0)

    @pl.when(i == 0)
    def _():
        pool_ref[:, :] = jnp.full_like(pool_ref, -jnp.inf)

    xb = x_ref[:, :]
    bfr = pn_ref[:, 6:7]
    for g in range(G):
        m = jnp.max(jnp.where(bfr == jnp.float32(g), xb, -jnp.inf), axis=0,
                    keepdims=True)
        pool_ref[g:g + 1, :] = jnp.maximum(pool_ref[g:g + 1, :], m)

    @pl.when(i == nb - 1)
    def _():
        o_ref[:, :] = (jnp.dot(pool_ref[:, :], wc_ref[:, :],
                               preferred_element_type=jnp.float32)
                       + bc_ref[:, :])


def _pool(x, pn, wc, bc):
    npad = x.shape[0]
    nb = npad // RB
    h = x.shape[1]
    c = wc.shape[1]
    return pl.pallas_call(
        functools.partial(_pool_body, nb),
        grid=(nb,),
        in_specs=[
            pl.BlockSpec((RB, h), lambda i: (i, 0)),
            pl.BlockSpec((RB, 16), lambda i: (i, 0)),
            pl.BlockSpec((h, c), lambda i: (0, 0)),
            pl.BlockSpec((1, c), lambda i: (0, 0)),
        ],
        out_specs=pl.BlockSpec((G, c), lambda i: (0, 0)),
        out_shape=jax.ShapeDtypeStruct((G, c), jnp.float32),
        scratch_shapes=[pltpu.VMEM((G, h), jnp.float32)],
    )(x, pn, wc, bc)


def kernel(pos, normal, batch, W1_0, b1_0, W2_0, b2_0, W1_1, b1_1, W2_1,
           b2_1, W1_2, b1_2, W2_2, b2_2, Wc, bc):
    n = pos.shape[0]
    npad = -(-n // CT) * CT
    pad = npad - n
    h = W2_0.shape[0]

    posp = jnp.pad(pos, ((0, pad), (0, 0)))
    normp = jnp.pad(normal, ((0, pad), (0, 0)))
    bf = jnp.pad(batch.astype(jnp.float32), (0, pad), constant_values=float(G))
    sq = jnp.sum(posp * posp, axis=1)
    pn = jnp.concatenate(
        [posp, normp, bf[:, None], sq[:, None],
         jnp.zeros((npad, 8), jnp.float32)], axis=1)
    pnT = pn.T

    idx = _knn(pn, pnT)
    idx3 = idx.T.reshape(NW, (K * npad) // (NW * RB), RB)

    pn128 = jnp.pad(pn, ((0, 0), (0, 128 - pn.shape[1])))
    png = _sc_gather(pn128, idx3).reshape(K, npad, 128)
    x, ppfT = _layer0(pn, png, W1_0, b1_0[None, :], W2_0, b2_0[None, :])

    for (w1, b1, w2, b2) in ((W1_1, b1_1, W2_1, b2_1),
                             (W1_2, b1_2, W2_2, b2_2)):
        xg = _sc_gather(x, idx3).reshape(K, npad, h)
        x = _layer(x, xg, ppfT, w1[:h], w1[h:], b1[None, :], w2, b2[None, :])

    return _pool(x, pn, Wc, bc[None, :])
